# baseline (speedup 1.0000x reference)
import functools

import jax
import jax.numpy as jnp
import numpy as np
from jax import lax
from jax.experimental import pallas as pl
from jax.experimental.pallas import tpu as pltpu
from jax.experimental.pallas import tpu_sc as plsc

B = 1
S = 2048
D = 768
H = 12
HD = 64
HH = HD // 2
E = 8
TOPK = 2
FFN = 768
BASE = 10000.0
EPS = 1e-5

ST = 256
QT = 512
NEG = jnp.finfo(jnp.float32).min
BF = jnp.bfloat16
F32 = jnp.float32


def _ln(x, scale):
    mu = jnp.mean(x, axis=-1, keepdims=True)
    var = jnp.mean(jnp.square(x - mu), axis=-1, keepdims=True)
    return (x - mu) / jnp.sqrt(var + EPS) * scale


def _qkv_body(x_ref, s1_ref, wq_ref, wk_ref, wv_ref, cos_ref, sin_ref,
              q_ref, k_ref, v_ref):
    h = _ln(x_ref[...], s1_ref[...]).astype(BF)
    cos = cos_ref[...]
    sin = sin_ref[...]

    q = jnp.dot(h, wq_ref[...], preferred_element_type=F32)
    q1 = q[:, : H * HH]
    q2 = q[:, H * HH:]
    scale = 1.0 / np.sqrt(HD)
    q_ref[:, : H * HH] = ((q1 * cos - q2 * sin) * scale).astype(BF)
    q_ref[:, H * HH:] = ((q2 * cos + q1 * sin) * scale).astype(BF)

    k = jnp.dot(h, wk_ref[...], preferred_element_type=F32)
    k1 = k[:, : H * HH]
    k2 = k[:, H * HH:]
    k_ref[:, : H * HH] = (k1 * cos - k2 * sin).astype(BF)
    k_ref[:, H * HH:] = (k2 * cos + k1 * sin).astype(BF)

    v_ref[...] = jnp.dot(h, wv_ref[...], preferred_element_type=F32).astype(BF)


def _attn_body(q_ref, k_ref, v_ref, o_ref):
    i = pl.program_id(0)
    qpos = i * QT + jax.lax.broadcasted_iota(jnp.int32, (QT, S), 0)
    kpos = jax.lax.broadcasted_iota(jnp.int32, (QT, S), 1)
    causal = kpos <= qpos
    for h in range(H):
        q1 = q_ref[:, HH * h: HH * h + HH]
        q2 = q_ref[:, H * HH + HH * h: H * HH + HH * h + HH]
        k1 = k_ref[:, HH * h: HH * h + HH]
        k2 = k_ref[:, H * HH + HH * h: H * HH + HH * h + HH]
        dn = (((1,), (1,)), ((), ()))
        s = jax.lax.dot_general(q1, k1, dn, preferred_element_type=F32)
        s = s + jax.lax.dot_general(q2, k2, dn, preferred_element_type=F32)
        s = jnp.where(causal, s, NEG)
        m = jnp.max(s, axis=1, keepdims=True)
        p = jnp.exp(s - m)
        l = jnp.sum(p, axis=1, keepdims=True)
        vh = v_ref[:, HD * h: HD * h + HD]
        pv = jnp.dot(p.astype(BF), vh, preferred_element_type=F32)
        o_ref[:, HD * h: HD * h + HD] = (pv / l).astype(BF)


NT = 24
MT = 256
MBUF = NT * MT


def _router_body(a_ref, wo_ref, x_ref, s2_ref, rw_ref,
                 r2_ref, h2_ref, w_ref, oh1_ref, oh2_ref, t1_ref, t2_ref):
    attn = jnp.dot(a_ref[...], wo_ref[...], preferred_element_type=F32)
    resid2 = x_ref[...] + attn
    r2_ref[...] = resid2
    h2 = _ln(resid2, s2_ref[...])
    h2_ref[...] = h2
    logits = jnp.dot(h2, rw_ref[...], preferred_element_type=F32)
    m = jnp.max(logits, axis=1, keepdims=True)
    ew = jnp.exp(logits - m)
    w = ew / jnp.sum(ew, axis=1, keepdims=True)
    w_ref[...] = w
    eidx = jax.lax.broadcasted_iota(jnp.int32, (ST, E), 1)
    e1 = jnp.argmax(w, axis=1)[:, None]
    oh1 = eidx == e1
    m1 = jnp.max(w, axis=1, keepdims=True)
    masked = jnp.where(oh1, -1.0, w)
    e2 = jnp.argmax(masked, axis=1)[:, None]
    oh2 = eidx == e2
    m2 = jnp.max(masked, axis=1, keepdims=True)
    denom = m1 + m2
    oh1_ref[...] = oh1.astype(F32)
    oh2_ref[...] = oh2.astype(F32)
    t1_ref[...] = m1 / denom
    t2_ref[...] = m2 / denom


def _meta_body(oh1_ref, oh2_ref, ltri_ref, u8_ref, r1_ref, r2_ref, et_ref):
    oh1 = oh1_ref[...]
    oh2 = oh2_ref[...]
    A = oh1 + oh2
    excl = jax.lax.dot_general(ltri_ref[...], A.astype(BF),
                               (((1,), (0,)), ((), ())),
                               preferred_element_type=F32)
    counts = excl[S - 1: S, :] + A[S - 1: S, :]
    padded = jnp.ceil(counts * (1.0 / MT)) * float(MT)
    off = jnp.dot(padded, u8_ref[...], preferred_element_type=F32)
    dest1 = jnp.sum(oh1 * (off + excl), axis=1, keepdims=True)
    dest2 = jnp.sum(oh2 * (off + excl + oh1), axis=1, keepdims=True)
    r1_ref[...] = dest1.astype(jnp.int32)
    r2_ref[...] = dest2.astype(jnp.int32)
    total = jnp.sum(padded, axis=1, keepdims=True)
    tpos = float(MT) * jax.lax.broadcasted_iota(jnp.int32, (32, 1), 0).astype(F32)
    cond = (tpos >= off) & (tpos < off + padded)
    eids = jax.lax.broadcasted_iota(jnp.int32, (32, E), 1).astype(F32)
    et = jnp.sum(jnp.where(cond, eids, 0.0), axis=1, keepdims=True)
    et = et + jnp.where(tpos >= total, float(E - 1), 0.0)
    et_ref[...] = et.astype(jnp.int32)


def _gffn_body(et_ref, x_ref, wg_ref, w1_ref, v1_ref, w2_ref, y_ref):
    del et_ref
    x = x_ref[...].astype(BF)
    dn_t = (((1,), (1,)), ((), ()))
    x1 = jax.lax.dot_general(x, w1_ref[...], dn_t, preferred_element_type=F32)
    x2 = jax.lax.dot_general(x, v1_ref[...], dn_t, preferred_element_type=F32)
    gl = (x1 * jax.nn.sigmoid(x1) * x2).astype(BF)
    y = jnp.dot(gl, w2_ref[...], preferred_element_type=F32)
    y_ref[...] = y * wg_ref[...]


NC = 2
NSUB = 16
NW = NC * NSUB
TPW = S // NW
CH = 32

_SC_MESH = plsc.VectorSubcoreMesh(core_axis_name="c", subcore_axis_name="s")


@functools.partial(
    pl.kernel,
    mesh=_SC_MESH,
    out_type=[
        jax.ShapeDtypeStruct((MBUF, D), F32),
        jax.ShapeDtypeStruct((MBUF,), F32),
    ],
    scratch_types=[
        pltpu.VMEM((TPW, D), F32),
        pltpu.VMEM((TPW,), jnp.int32),
        pltpu.VMEM((TPW,), jnp.int32),
        pltpu.VMEM((TPW,), F32),
        pltpu.VMEM((TPW,), F32),
        pltpu.SemaphoreType.DMA,
    ],
)
def _sc_dispatch(h2_hbm, r1_hbm, r2_hbm, t1_hbm, t2_hbm,
                 xg_hbm, wgt_hbm, hv, i1, i2, w1v, w2v, sem):
    wid = lax.axis_index("s") * NC + lax.axis_index("c")
    base = wid * TPW
    pltpu.sync_copy(r1_hbm.at[pl.ds(base, TPW)], i1)
    pltpu.sync_copy(r2_hbm.at[pl.ds(base, TPW)], i2)
    pltpu.sync_copy(t1_hbm.at[pl.ds(base, TPW)], w1v)
    pltpu.sync_copy(t2_hbm.at[pl.ds(base, TPW)], w2v)
    pltpu.sync_copy(h2_hbm.at[pl.ds(base, TPW)], hv)
    pltpu.async_copy(hv, xg_hbm.at[i1], sem).wait()
    pltpu.async_copy(hv, xg_hbm.at[i2], sem).wait()
    pltpu.async_copy(w1v, wgt_hbm.at[i1], sem).wait()
    pltpu.async_copy(w2v, wgt_hbm.at[i2], sem).wait()


@functools.partial(
    pl.kernel,
    mesh=_SC_MESH,
    out_type=jax.ShapeDtypeStruct((S, D), F32),
    scratch_types=[
        pltpu.VMEM((TPW,), jnp.int32),
        pltpu.VMEM((TPW,), jnp.int32),
        pltpu.VMEM((CH, D), F32),
        pltpu.VMEM((CH, D), F32),
        pltpu.VMEM((CH, D), F32),
        pltpu.SemaphoreType.DMA,
    ],
)
def _sc_combine(yg_hbm, r1_hbm, r2_hbm, res_hbm, out_hbm,
                i1, i2, y1v, y2v, rv, sem):
    wid = lax.axis_index("s") * NC + lax.axis_index("c")
    base = wid * TPW
    pltpu.sync_copy(r1_hbm.at[pl.ds(base, TPW)], i1)
    pltpu.sync_copy(r2_hbm.at[pl.ds(base, TPW)], i2)
    for b in range(TPW // CH):
        pltpu.async_copy(yg_hbm.at[i1.at[pl.ds(b * CH, CH)]], y1v, sem).wait()
        pltpu.async_copy(yg_hbm.at[i2.at[pl.ds(b * CH, CH)]], y2v, sem).wait()
        pltpu.sync_copy(res_hbm.at[pl.ds(base + b * CH, CH)], rv)

        def _row(r, _):
            def _col(cb, _):
                for u in range(8):
                    cs = cb * 128 + u * 16
                    rv[r, pl.ds(cs, 16)] = (rv[r, pl.ds(cs, 16)]
                                            + y1v[r, pl.ds(cs, 16)]
                                            + y2v[r, pl.ds(cs, 16)])
                return 0

            lax.fori_loop(0, D // 128, _col, 0)
            return 0

        lax.fori_loop(0, CH, _row, 0)
        pltpu.sync_copy(rv, out_hbm.at[pl.ds(base + b * CH, CH)])


def _build_tables():
    inv_freq = 1.0 / (BASE ** (np.arange(0, HD, 2, dtype=np.float32) / HD))
    pos = np.arange(S, dtype=np.float32)
    freqs = pos[:, None] * inv_freq[None, :]
    cos = np.tile(np.cos(freqs), (1, H)).astype(np.float32)
    sin = np.tile(np.sin(freqs), (1, H)).astype(np.float32)
    perm = np.concatenate([
        np.concatenate([np.arange(HH) + HD * h for h in range(H)]),
        np.concatenate([np.arange(HH) + HH + HD * h for h in range(H)]),
    ])
    return cos, sin, perm


_COS, _SIN, _PERM = _build_tables()
_LTRI = np.tril(np.ones((S, S), dtype=np.float32), k=-1)
_U8 = np.triu(np.ones((E, E), dtype=np.float32), k=1)


@jax.jit
def kernel(x, ln1_scale, ln2_scale, Wqkv, Wout, router_w, w1, v1, w2):
    xf = x.reshape(S, D)
    wq = Wqkv[:, :D][:, _PERM].astype(BF)
    wk = Wqkv[:, D:2 * D][:, _PERM].astype(BF)
    wv = Wqkv[:, 2 * D:].astype(BF)
    wo = Wout.astype(BF)
    w1b = w1.astype(BF)
    v1b = v1.astype(BF)
    w2b = w2.astype(BF)
    s1 = ln1_scale.reshape(1, D)
    s2 = ln2_scale.reshape(1, D)

    nst = S // ST

    q, k, v = pl.pallas_call(
        _qkv_body,
        grid=(nst,),
        in_specs=[
            pl.BlockSpec((ST, D), lambda i: (i, 0)),
            pl.BlockSpec((1, D), lambda i: (0, 0)),
            pl.BlockSpec((D, D), lambda i: (0, 0)),
            pl.BlockSpec((D, D), lambda i: (0, 0)),
            pl.BlockSpec((D, D), lambda i: (0, 0)),
            pl.BlockSpec((ST, H * HH), lambda i: (i, 0)),
            pl.BlockSpec((ST, H * HH), lambda i: (i, 0)),
        ],
        out_specs=[
            pl.BlockSpec((ST, D), lambda i: (i, 0)),
            pl.BlockSpec((ST, D), lambda i: (i, 0)),
            pl.BlockSpec((ST, D), lambda i: (i, 0)),
        ],
        out_shape=[jax.ShapeDtypeStruct((S, D), BF)] * 3,
    )(xf, s1, wq, wk, wv, _COS, _SIN)

    attn = pl.pallas_call(
        _attn_body,
        grid=(S // QT,),
        in_specs=[
            pl.BlockSpec((QT, D), lambda i: (i, 0)),
            pl.BlockSpec((S, D), lambda i: (0, 0)),
            pl.BlockSpec((S, D), lambda i: (0, 0)),
        ],
        out_specs=pl.BlockSpec((QT, D), lambda i: (i, 0)),
        out_shape=jax.ShapeDtypeStruct((S, D), BF),
    )(q, k, v)

    resid2, h2, weights, oh1f, oh2f, tw1, tw2 = pl.pallas_call(
        _router_body,
        grid=(nst,),
        in_specs=[
            pl.BlockSpec((ST, D), lambda i: (i, 0)),
            pl.BlockSpec((D, D), lambda i: (0, 0)),
            pl.BlockSpec((ST, D), lambda i: (i, 0)),
            pl.BlockSpec((1, D), lambda i: (0, 0)),
            pl.BlockSpec((D, E), lambda i: (0, 0)),
        ],
        out_specs=[
            pl.BlockSpec((ST, D), lambda i: (i, 0)),
            pl.BlockSpec((ST, D), lambda i: (i, 0)),
            pl.BlockSpec((ST, E), lambda i: (i, 0)),
            pl.BlockSpec((ST, E), lambda i: (i, 0)),
            pl.BlockSpec((ST, E), lambda i: (i, 0)),
            pl.BlockSpec((ST, 1), lambda i: (i, 0)),
            pl.BlockSpec((ST, 1), lambda i: (i, 0)),
        ],
        out_shape=[
            jax.ShapeDtypeStruct((S, D), F32),
            jax.ShapeDtypeStruct((S, D), F32),
            jax.ShapeDtypeStruct((S, E), F32),
            jax.ShapeDtypeStruct((S, E), F32),
            jax.ShapeDtypeStruct((S, E), F32),
            jax.ShapeDtypeStruct((S, 1), F32),
            jax.ShapeDtypeStruct((S, 1), F32),
        ],
    )(attn, wo, xf, s2, router_w)

    r1, r2, etile = pl.pallas_call(
        _meta_body,
        in_specs=[
            pl.BlockSpec((S, E), lambda: (0, 0)),
            pl.BlockSpec((S, E), lambda: (0, 0)),
            pl.BlockSpec((S, S), lambda: (0, 0)),
            pl.BlockSpec((E, E), lambda: (0, 0)),
        ],
        out_specs=[
            pl.BlockSpec((S, 1), lambda: (0, 0)),
            pl.BlockSpec((S, 1), lambda: (0, 0)),
            pl.BlockSpec((32, 1), lambda: (0, 0)),
        ],
        out_shape=[
            jax.ShapeDtypeStruct((S, 1), jnp.int32),
            jax.ShapeDtypeStruct((S, 1), jnp.int32),
            jax.ShapeDtypeStruct((32, 1), jnp.int32),
        ],
    )(oh1f, oh2f, jnp.asarray(_LTRI).astype(BF), jnp.asarray(_U8))

    r1f = r1.reshape(S)
    r2f = r2.reshape(S)
    xg, wgt = _sc_dispatch(h2, r1f, r2f, tw1.reshape(S), tw2.reshape(S))
    wgt = wgt.reshape(MBUF, 1)

    yg = pl.pallas_call(
        _gffn_body,
        grid_spec=pltpu.PrefetchScalarGridSpec(
            num_scalar_prefetch=1,
            grid=(NT,),
            in_specs=[
                pl.BlockSpec((MT, D), lambda j, et: (j, 0)),
                pl.BlockSpec((MT, 1), lambda j, et: (j, 0)),
                pl.BlockSpec((FFN, D), lambda j, et: (et[j], 0)),
                pl.BlockSpec((FFN, D), lambda j, et: (et[j], 0)),
                pl.BlockSpec((FFN, D), lambda j, et: (et[j], 0)),
            ],
            out_specs=pl.BlockSpec((MT, D), lambda j, et: (j, 0)),
        ),
        out_shape=jax.ShapeDtypeStruct((MBUF, D), F32),
    )(etile.reshape(32), xg, wgt, w1b, v1b, w2b)

    out = _sc_combine(yg, r1f, r2f, resid2)
    return out.reshape(B, S, D), weights.reshape(B, S, E)
